# baseline (speedup 1.0000x reference)
import functools

import jax
import jax.numpy as jnp
from jax import lax
from jax.experimental import pallas as pl
from jax.experimental.pallas import tpu as pltpu
from jax.experimental.pallas import tpu_sc as plsc

_INFO = plsc.get_sparse_core_info()
_NC = _INFO.num_cores
_NS = _INFO.num_subcores
_NW = _NC * _NS
_LANES = _INFO.num_lanes
_UNROLL = 8
_NSTR = 4
_CHUNK = 2048
_TILE = 128


def _make_gather(d: int, vocab: int, n_idx: int):
    rows_per_w = d // _NW
    n_tiles = vocab // _TILE
    main = n_tiles * _TILE
    tail = vocab - main
    row_words = main + (_TILE if tail else 0)
    n_chunks = n_idx // _CHUNK
    gpc = _CHUNK // (_LANES * _UNROLL)
    mesh = plsc.VectorSubcoreMesh(core_axis_name="c", subcore_axis_name="s")

    @functools.partial(
        pl.kernel,
        out_type=jax.ShapeDtypeStruct((2, d, n_idx), jnp.float32),
        mesh=mesh,
        scratch_types=[
            pltpu.VMEM((row_words,), jnp.float32),
            pltpu.VMEM((n_idx,), jnp.int32),
            pltpu.VMEM((n_idx,), jnp.int32),
            pltpu.VMEM((2, _CHUNK), jnp.float32),
            pltpu.SemaphoreType.DMA,
            pltpu.SemaphoreType.DMA,
            pltpu.SemaphoreType.DMA,
        ],
        compiler_params=pltpu.CompilerParams(needs_layout_passes=False),
    )
    def gather_kernel(idx_text_hbm, idx_tags_hbm, table_t_hbm, tail_t_hbm,
                      out_hbm, row_v, idx_text_v, idx_tags_v, out_v,
                      sem_row, sem_idx, sem_out):
        wid = lax.axis_index("s") * _NC + lax.axis_index("c")

        def start_row(comp):
            hs = []
            lo = 0
            q, rem = divmod(n_tiles, _NSTR)
            for s in range(_NSTR):
                sz = (q + (1 if s < rem else 0)) * _TILE
                if sz:
                    hs.append(pltpu.async_copy(
                        table_t_hbm.at[comp, pl.ds(lo, sz)],
                        row_v.at[pl.ds(lo, sz)], sem_row))
                    lo += sz
            if tail:
                hs.append(pltpu.async_copy(
                    tail_t_hbm.at[comp],
                    row_v.at[pl.ds(main, _TILE)], sem_row))
            return hs

        h_text = pltpu.async_copy(idx_text_hbm, idx_text_v, sem_idx)
        h_tags = pltpu.async_copy(idx_tags_hbm, idx_tags_v, sem_idx)
        h_row = start_row(wid * rows_per_w)
        h_text.wait()
        h_tags.wait()

        pend = [None, None]
        for r in range(rows_per_w):
            comp = wid * rows_per_w + r
            for h in h_row:
                h.wait()
            for t, idx_v in ((0, idx_text_v), (1, idx_tags_v)):
                for c in range(n_chunks):
                    b = (t * n_chunks + c) % 2
                    if pend[b] is not None:
                        pend[b].wait()
                    base = c * _CHUNK

                    def body(g, _, idx_v=idx_v, b=b, base=base):
                        for j in range(_UNROLL):
                            off = (g * _UNROLL + j) * _LANES
                            iv = idx_v[pl.ds(base + off, _LANES)]
                            out_v[b, pl.ds(off, _LANES)] = plsc.load_gather(
                                row_v, [iv])
                        return 0

                    lax.fori_loop(0, gpc, body, 0)
                    pend[b] = pltpu.async_copy(
                        out_v.at[b],
                        out_hbm.at[t, comp, pl.ds(base, _CHUNK)], sem_out)
            if r + 1 < rows_per_w:
                h_row = start_row(comp + 1)
        for p in pend:
            if p is not None:
                p.wait()

    return gather_kernel


def kernel(text_tokens, predictions, tag_vocab):
    L = text_tokens.shape[0]
    vocab, d = tag_vocab.shape
    slice_tags = predictions[0, -L:]
    main = (vocab // _TILE) * _TILE
    tail_rows = jnp.zeros((_TILE, d), jnp.float32).at[: vocab - main].set(
        tag_vocab[main:, :])
    out_t = _make_gather(d, vocab, L)(
        text_tokens.astype(jnp.int32),
        slice_tags.astype(jnp.int32),
        tag_vocab.T,
        jnp.transpose(tail_rows),
    )
    return jnp.transpose(out_t, (2, 0, 1))


# speedup vs baseline: 1.2341x; 1.2341x over previous
import functools

import jax
import jax.numpy as jnp
from jax import lax
from jax.experimental import pallas as pl
from jax.experimental.pallas import tpu as pltpu
from jax.experimental.pallas import tpu_sc as plsc

_INFO = plsc.get_sparse_core_info()
_NC = _INFO.num_cores
_NS = _INFO.num_subcores
_NW = _NC * _NS
_LANES = _INFO.num_lanes
_UNROLL = 8
_NSTR = 4
_CHUNK = 2048
_TILE = 128


def _make_gather(d: int, vocab: int, n_idx: int):
    rows_per_w = d // _NW
    n_tiles = vocab // _TILE
    main = n_tiles * _TILE
    tail = vocab - main
    row_words = main + (_TILE if tail else 0)
    n_chunks = n_idx // _CHUNK
    gpc = _CHUNK // (_LANES * _UNROLL)
    mesh = plsc.VectorSubcoreMesh(core_axis_name="c", subcore_axis_name="s")

    @functools.partial(
        pl.kernel,
        out_type=jax.ShapeDtypeStruct((2, d, n_idx), jnp.float32),
        mesh=mesh,
        scratch_types=[
            pltpu.VMEM((row_words,), jnp.float32),
            pltpu.VMEM((n_idx,), jnp.int32),
            pltpu.VMEM((n_idx,), jnp.int32),
            pltpu.VMEM((n_idx,), jnp.float32),
            pltpu.SemaphoreType.DMA,
            pltpu.SemaphoreType.DMA,
            pltpu.SemaphoreType.DMA,
        ],
        compiler_params=pltpu.CompilerParams(needs_layout_passes=False),
    )
    def gather_kernel(idx_text_hbm, idx_tags_hbm, table_t_hbm, tail_t_hbm,
                      out_hbm, row_v, idx_text_v, idx_tags_v, out_v,
                      sem_row, sem_idx, sem_out):
        wid = lax.axis_index("s") * _NC + lax.axis_index("c")

        def start_row(comp):
            hs = []
            lo = 0
            q, rem = divmod(n_tiles, _NSTR)
            for s in range(_NSTR):
                sz = (q + (1 if s < rem else 0)) * _TILE
                if sz:
                    hs.append(pltpu.async_copy(
                        table_t_hbm.at[comp, pl.ds(lo, sz)],
                        row_v.at[pl.ds(lo, sz)], sem_row))
                    lo += sz
            if tail:
                hs.append(pltpu.async_copy(
                    tail_t_hbm.at[comp],
                    row_v.at[pl.ds(main, _TILE)], sem_row))
            return hs

        h_text = pltpu.async_copy(idx_text_hbm, idx_text_v, sem_idx)
        h_tags = pltpu.async_copy(idx_tags_hbm, idx_tags_v, sem_idx)
        h_row = start_row(wid * rows_per_w)
        h_text.wait()
        h_tags.wait()

        n_groups = n_idx // (_LANES * _UNROLL)
        for r in range(rows_per_w):
            comp = wid * rows_per_w + r
            for h in h_row:
                h.wait()
            for t, idx_v in ((0, idx_text_v), (1, idx_tags_v)):

                def body(g, _, idx_v=idx_v):
                    for j in range(_UNROLL):
                        off = (g * _UNROLL + j) * _LANES
                        iv = idx_v[pl.ds(off, _LANES)]
                        out_v[pl.ds(off, _LANES)] = plsc.load_gather(
                            row_v, [iv])
                    return 0

                lax.fori_loop(0, n_groups, body, 0)
                pltpu.sync_copy(out_v, out_hbm.at[t, comp])
            if r + 1 < rows_per_w:
                h_row = start_row(comp + 1)

    return gather_kernel


def kernel(text_tokens, predictions, tag_vocab):
    L = text_tokens.shape[0]
    vocab, d = tag_vocab.shape
    slice_tags = predictions[0, -L:]
    main = (vocab // _TILE) * _TILE
    tail_rows = jnp.zeros((_TILE, d), jnp.float32).at[: vocab - main].set(
        tag_vocab[main:, :])
    out_t = _make_gather(d, vocab, L)(
        text_tokens.astype(jnp.int32),
        slice_tags.astype(jnp.int32),
        tag_vocab.T,
        jnp.transpose(tail_rows),
    )
    return jnp.transpose(out_t, (2, 0, 1))


# speedup vs baseline: 1.2380x; 1.0032x over previous
import functools

import jax
import jax.numpy as jnp
from jax import lax
from jax.experimental import pallas as pl
from jax.experimental.pallas import tpu as pltpu
from jax.experimental.pallas import tpu_sc as plsc

_INFO = plsc.get_sparse_core_info()
_NC = _INFO.num_cores
_NS = _INFO.num_subcores
_NW = _NC * _NS
_LANES = _INFO.num_lanes
_UNROLL = 16


def _make_gather(d: int, vocab: int, n_idx: int):
    rows_per_w = d // _NW
    n_groups = n_idx // (_LANES * _UNROLL)
    mesh = plsc.VectorSubcoreMesh(core_axis_name="c", subcore_axis_name="s")

    @functools.partial(
        pl.kernel,
        out_type=jax.ShapeDtypeStruct((2, d, n_idx), jnp.float32),
        mesh=mesh,
        scratch_types=[
            pltpu.VMEM((vocab,), jnp.float32),
            pltpu.VMEM((n_idx,), jnp.int32),
            pltpu.VMEM((n_idx,), jnp.int32),
            pltpu.VMEM((n_idx,), jnp.float32),
            pltpu.SemaphoreType.DMA,
            pltpu.SemaphoreType.DMA,
        ],
        compiler_params=pltpu.CompilerParams(needs_layout_passes=False),
    )
    def gather_kernel(idx_text_hbm, idx_tags_hbm, table_t_hbm, out_hbm,
                      row_v, idx_text_v, idx_tags_v, out_v,
                      sem_row, sem_idx):
        wid = lax.axis_index("s") * _NC + lax.axis_index("c")
        h_text = pltpu.async_copy(idx_text_hbm, idx_text_v, sem_idx)
        h_tags = pltpu.async_copy(idx_tags_hbm, idx_tags_v, sem_idx)
        h_row = pltpu.async_copy(table_t_hbm.at[wid * rows_per_w], row_v,
                                 sem_row)
        h_text.wait()
        h_tags.wait()

        for r in range(rows_per_w):
            comp = wid * rows_per_w + r
            h_row.wait()
            for t, idx_v in ((0, idx_text_v), (1, idx_tags_v)):

                def body(g, _, idx_v=idx_v):
                    for j in range(_UNROLL):
                        off = (g * _UNROLL + j) * _LANES
                        iv = idx_v[pl.ds(off, _LANES)]
                        out_v[pl.ds(off, _LANES)] = plsc.load_gather(
                            row_v, [iv])
                    return 0

                lax.fori_loop(0, n_groups, body, 0)
                if t == 1 and r + 1 < rows_per_w:
                    h_row = pltpu.async_copy(
                        table_t_hbm.at[comp + 1], row_v, sem_row)
                pltpu.sync_copy(out_v, out_hbm.at[t, comp])

    return gather_kernel


def kernel(text_tokens, predictions, tag_vocab):
    L = text_tokens.shape[0]
    vocab, d = tag_vocab.shape
    slice_tags = predictions[0, -L:]
    out_t = _make_gather(d, vocab, L)(
        text_tokens.astype(jnp.int32),
        slice_tags.astype(jnp.int32),
        tag_vocab.T,
    )
    return jnp.transpose(out_t, (2, 0, 1))
